# Initial kernel scaffold; baseline (speedup 1.0000x reference)
import jax
import jax.numpy as jnp
from jax.experimental import pallas as pl


def kernel(pc, mask):
    raise NotImplementedError("write your pallas kernel here")


# speedup vs baseline: 20.0488x; 20.0488x over previous
import jax
import jax.numpy as jnp
from jax.experimental import pallas as pl
from jax.experimental.pallas import tpu as pltpu

K_BALL = 16
RADIUS2 = 0.2 * 0.2
TN = 256
TM = 512


def _inclusive_cumsum_lanes(x):
    n = x.shape[-1]
    s = 1
    while s < n:
        x = x + jnp.pad(x[:, :-s], ((0, 0), (s, 0)))
        s *= 2
    return x


def _body(pc_ref, mask_ref, pct_ref, maskt_ref, out_ref):
    b = pl.program_id(0)
    i = pl.program_id(1)

    pcb = pc_ref[0]
    maskb = mask_ref[0]
    pct = pct_ref[0]
    maskt = maskt_ref[0]

    n_total = pct.shape[1]

    cnt = jnp.zeros((TN, 1), jnp.int32)
    acc = jnp.zeros((1, 1), jnp.float32)

    for c0 in range(0, n_total, TM):
        d2 = jnp.zeros((TN, TM), jnp.float32)
        for c in range(3):
            diff = pcb[:, c:c + 1] - pct[c:c + 1, c0:c0 + TM]
            d2 = d2 + diff * diff
        within = d2 < RADIUS2
        wi = within.astype(jnp.int32)
        incl = _inclusive_cumsum_lanes(wi)
        rank = cnt + (incl - wi)
        sel = within & (rank < K_BALL)

        l1 = jnp.zeros((TN, TM), jnp.float32)
        for c in range(16):
            l1 = l1 + jnp.abs(maskb[:, c:c + 1] - maskt[c:c + 1, c0:c0 + TM])

        acc = acc + jnp.sum(jnp.where(sel, l1, 0.0))
        cnt = cnt + incl[:, -1:]

    @pl.when((b == 0) & (i == 0))
    def _init():
        out_ref[...] = jnp.zeros_like(out_ref)

    out_ref[...] += acc


def kernel(pc, mask):
    B, N, _ = pc.shape
    pct = jnp.transpose(pc, (0, 2, 1))
    maskt = jnp.transpose(mask, (0, 2, 1))
    total = pl.pallas_call(
        _body,
        grid=(B, N // TN),
        in_specs=[
            pl.BlockSpec((1, TN, 3), lambda b, i: (b, i, 0)),
            pl.BlockSpec((1, TN, 16), lambda b, i: (b, i, 0)),
            pl.BlockSpec((1, 3, N), lambda b, i: (b, 0, 0)),
            pl.BlockSpec((1, 16, N), lambda b, i: (b, 0, 0)),
        ],
        out_specs=pl.BlockSpec((1, 1), lambda b, i: (0, 0)),
        out_shape=jax.ShapeDtypeStruct((1, 1), jnp.float32),
        compiler_params=pltpu.CompilerParams(
            dimension_semantics=("arbitrary", "arbitrary")),
    )(pc, mask, pct, maskt)
    return total[0, 0] / (B * N * K_BALL)


# speedup vs baseline: 42.1516x; 2.1024x over previous
import jax
import jax.numpy as jnp
from jax.experimental import pallas as pl
from jax.experimental.pallas import tpu as pltpu

K_BALL = 16
RADIUS2 = 0.2 * 0.2
TN = 256
TM = 512


def _body(pc_ref, mask_ref, pct_ref, maskt_ref, out_ref):
    b = pl.program_id(0)
    i = pl.program_id(1)

    pcb = pc_ref[0]
    pct = pct_ref[0]
    maskb = mask_ref[0].astype(jnp.bfloat16)
    maskt = maskt_ref[0].astype(jnp.bfloat16)

    n_total = pct.shape[1]

    rows = jax.lax.broadcasted_iota(jnp.int32, (TM, TM), 0)
    cols = jax.lax.broadcasted_iota(jnp.int32, (TM, TM), 1)
    tri = (rows <= cols).astype(jnp.bfloat16)

    cnt = jnp.zeros((TN, 1), jnp.float32)
    acc = jnp.zeros((1, 1), jnp.float32)

    for c0 in range(0, n_total, TM):
        d2 = jnp.zeros((TN, TM), jnp.float32)
        for c in range(3):
            diff = pcb[:, c:c + 1] - pct[c:c + 1, c0:c0 + TM]
            d2 = d2 + diff * diff
        within = d2 < RADIUS2
        wf = within.astype(jnp.float32)
        incl = jax.lax.dot_general(
            within.astype(jnp.bfloat16), tri,
            (((1,), (0,)), ((), ())),
            preferred_element_type=jnp.float32)
        rank = cnt + (incl - wf)
        sel = within & (rank < K_BALL)

        l1 = jnp.zeros((TN, TM), jnp.bfloat16)
        for c in range(16):
            l1 = l1 + jnp.abs(maskb[:, c:c + 1] - maskt[c:c + 1, c0:c0 + TM])

        contrib = jnp.where(sel, l1.astype(jnp.float32), 0.0)
        acc = acc + jnp.sum(contrib)
        cnt = cnt + incl[:, -1:]

    @pl.when((b == 0) & (i == 0))
    def _init():
        out_ref[...] = jnp.zeros_like(out_ref)

    out_ref[...] += acc


def kernel(pc, mask):
    B, N, _ = pc.shape
    pct = jnp.transpose(pc, (0, 2, 1))
    maskt = jnp.transpose(mask, (0, 2, 1))
    total = pl.pallas_call(
        _body,
        grid=(B, N // TN),
        in_specs=[
            pl.BlockSpec((1, TN, 3), lambda b, i: (b, i, 0)),
            pl.BlockSpec((1, TN, 16), lambda b, i: (b, i, 0)),
            pl.BlockSpec((1, 3, N), lambda b, i: (b, 0, 0)),
            pl.BlockSpec((1, 16, N), lambda b, i: (b, 0, 0)),
        ],
        out_specs=pl.BlockSpec((1, 1), lambda b, i: (0, 0)),
        out_shape=jax.ShapeDtypeStruct((1, 1), jnp.float32),
        compiler_params=pltpu.CompilerParams(
            dimension_semantics=("arbitrary", "arbitrary")),
    )(pc, mask, pct, maskt)
    return total[0, 0] / (B * N * K_BALL)
